# baseline (speedup 1.0000x reference)
import functools

import numpy as np
import jax
import jax.numpy as jnp
from jax import lax
from jax.experimental import pallas as pl
from jax.experimental.pallas import tpu as pltpu
from jax.experimental.pallas import tpu_sc as plsc

N = 10000
E = 320000
NG = 16
NODE = 128
EDGE = 64
HID = 128
NL = 3
MAXZ = 100
MAXZP = 104

NP = 10240
EP = 327680
NBLK = 1280
EBLK = 2560

NC = 2
NS = 16
NW = NC * NS
K = 128
PERW = EP // NW
NCHUNK = PERW // K
RPS = NP // NS


def _swish(v):
    return v * (1.0 / (1.0 + jnp.exp(-v)))


def _pre_body(az_ref, bat_ref, pos16_ref, emb_ref, latf_ref, w1a_ref, w1b_ref,
              be1_ref, x_ref, a_ref, b_ref, u_ref):
    az = az_ref[...]
    ioz = lax.broadcasted_iota(jnp.int32, (1, MAXZP), 1)
    oh = (az == ioz).astype(jnp.float32)
    x = jnp.dot(oh, emb_ref[...], preferred_element_type=jnp.float32)
    x_ref[...] = x
    a_ref[...] = jnp.dot(x, w1a_ref[...], preferred_element_type=jnp.float32) + be1_ref[...]
    b_ref[...] = jnp.dot(x, w1b_ref[...], preferred_element_type=jnp.float32)
    bat = bat_ref[...]
    iog = lax.broadcasted_iota(jnp.int32, (1, NG), 1)
    ohg = (bat == iog).astype(jnp.float32)
    u_ref[...] = pos16_ref[...] + jnp.dot(ohg, latf_ref[...],
                                          preferred_element_type=jnp.float32)


def _dsq_body(gs_ref, gd_ref, sh_ref, selq_ref, selr_ref, e3_ref, one3_ref,
              dsq_ref):
    gs = gs_ref[...]
    gd = gd_ref[...]
    shb = jnp.dot(sh_ref[...], selq_ref[...],
                  preferred_element_type=jnp.float32)
    full = (gd - gs) * e3_ref[...] + shb * gs
    ev = jnp.dot(full, selr_ref[...], preferred_element_type=jnp.float32)
    dsq_ref[...] = jnp.dot(ev * ev, one3_ref[...], preferred_element_type=jnp.float32)


def _mid_body(pq_ref, dsq_ref, w1c_ref, w2_ref, be2_ref, m_ref):
    z = _swish(pq_ref[...] + dsq_ref[...] * w1c_ref[...])
    m = jnp.dot(z, w2_ref[...], preferred_element_type=jnp.float32) + be2_ref[...]
    m_ref[...] = _swish(m)


def _node_body(x_ref, agg_ref, wn1x_ref, wn1a_ref, bn1_ref, wn2_ref, bn2_ref,
               wa_ref, ba_ref, wb_ref, xn_ref, a_ref, b_ref):
    agg = agg_ref[0] + agg_ref[1]
    h = _swish(jnp.dot(x_ref[...], wn1x_ref[...], preferred_element_type=jnp.float32)
               + jnp.dot(agg, wn1a_ref[...], preferred_element_type=jnp.float32)
               + bn1_ref[...])
    xn = x_ref[...] + jnp.dot(h, wn2_ref[...], preferred_element_type=jnp.float32) + bn2_ref[...]
    xn_ref[...] = xn
    a_ref[...] = jnp.dot(xn, wa_ref[...], preferred_element_type=jnp.float32) + ba_ref[...]
    b_ref[...] = jnp.dot(xn, wb_ref[...], preferred_element_type=jnp.float32)


def _read_body(a2_ref, bat_ref, wo2_ref, bo2_ref, out_ref):
    h = _swish(a2_ref[...])
    o = jnp.dot(h, wo2_ref[...], preferred_element_type=jnp.float32) + bo2_ref[...]
    iog = lax.broadcasted_iota(jnp.int32, (1, NG), 1)
    msk = (bat_ref[...] == iog).astype(jnp.float32)
    contrib = lax.dot_general(msk, o, (((0,), (0,)), ((), ())),
                              preferred_element_type=jnp.float32)
    @pl.when(pl.program_id(0) == 0)
    def _():
        out_ref[...] = jnp.zeros_like(out_ref)
    out_ref[...] += contrib


def _full(shape):
    return pl.BlockSpec(shape, lambda i: tuple(0 for _ in shape))


def _rows(shape):
    nd = len(shape)
    if nd == 2:
        return pl.BlockSpec(shape, lambda i: (i, 0))
    return pl.BlockSpec(shape, lambda i: (0, i, 0))


_pre_call = pl.pallas_call(
    _pre_body,
    grid=(NP // NBLK,),
    in_specs=[_rows((NBLK, 1)), _rows((NBLK, 1)), _rows((NBLK, 16)),
              _full((MAXZP, NODE)), _full((NG, 16)), _full((NODE, HID)),
              _full((NODE, HID)), _full((1, HID))],
    out_specs=[_rows((NBLK, NODE)), _rows((NBLK, HID)), _rows((NBLK, HID)),
               _rows((NBLK, 16))],
    out_shape=[jax.ShapeDtypeStruct((NP, NODE), jnp.float32),
               jax.ShapeDtypeStruct((NP, HID), jnp.float32),
               jax.ShapeDtypeStruct((NP, HID), jnp.float32),
               jax.ShapeDtypeStruct((NP, 16), jnp.float32)],
)

_dsq_call = pl.pallas_call(
    _dsq_body,
    grid=(EP // EBLK,),
    in_specs=[_rows((EBLK, 16)), _rows((EBLK, 16)), _rows((EBLK, 3)),
              _full((3, 16)), _full((16, 3)), _full((1, 16)), _full((3, 1))],
    out_specs=[_rows((EBLK, 1))],
    out_shape=[jax.ShapeDtypeStruct((EP, 1), jnp.float32)],
)

_mid_call = pl.pallas_call(
    _mid_body,
    grid=(EP // EBLK,),
    in_specs=[_rows((EBLK, HID)), _rows((EBLK, 1)), _full((1, HID)),
              _full((HID, EDGE)), _full((1, EDGE))],
    out_specs=[_rows((EBLK, EDGE))],
    out_shape=[jax.ShapeDtypeStruct((EP, EDGE), jnp.float32)],
)

_node_call = pl.pallas_call(
    _node_body,
    grid=(NP // NBLK,),
    in_specs=[_rows((NBLK, NODE)), _rows((NC, NBLK, EDGE)),
              _full((NODE, HID)), _full((EDGE, HID)), _full((1, HID)),
              _full((HID, NODE)), _full((1, NODE)),
              _full((NODE, HID)), _full((1, HID)), _full((NODE, HID))],
    out_specs=[_rows((NBLK, NODE)), _rows((NBLK, HID)), _rows((NBLK, HID))],
    out_shape=[jax.ShapeDtypeStruct((NP, NODE), jnp.float32),
               jax.ShapeDtypeStruct((NP, HID), jnp.float32),
               jax.ShapeDtypeStruct((NP, HID), jnp.float32)],
)

_read_call = pl.pallas_call(
    _read_body,
    grid=(NP // NBLK,),
    in_specs=[_rows((NBLK, HID)), _rows((NBLK, 1)), _full((HID, 1)),
              _full((1, 1))],
    out_specs=[_full((NG, 1))],
    out_shape=[jax.ShapeDtypeStruct((NG, 1), jnp.float32)],
)


_MESH = plsc.VectorSubcoreMesh(core_axis_name="c", subcore_axis_name="s")
_SC_PARAMS = pltpu.CompilerParams(use_tc_tiling_on_sc=False)


def _worker_id():
    return lax.axis_index("s") * NC + lax.axis_index("c")


NBUF = 4
NBUF_S = 5
NGRP = NCHUNK // NBUF


def _wait(src, dst, sem):
    pltpu.make_async_copy(src, dst, sem).wait()


def _geom_body(u_hbm, src3_hbm, dst3_hbm, gs_hbm, gd_hbm, sidx, didx,
               bufs0, bufs1, bufs2, bufs3, bufd0, bufd1, bufd2, bufd3,
               semS, semD, semWS, semWD):
    wid = _worker_id()
    base0 = wid * PERW
    bufs = [bufs0, bufs1, bufs2, bufs3]
    bufd = [bufd0, bufd1, bufd2, bufd3]
    pltpu.sync_copy(src3_hbm.at[wid], sidx)
    pltpu.sync_copy(dst3_hbm.at[wid], didx)
    for k in range(NBUF):
        pltpu.async_copy(u_hbm.at[sidx.at[k]], bufs[k], semS.at[k])
        pltpu.async_copy(u_hbm.at[didx.at[k]], bufd[k], semD.at[k])

    def group(g, carry):
        for k in range(NBUF):
            j = g * NBUF + k
            base = base0 + j * K
            _wait(u_hbm.at[sidx.at[0]], bufs[k], semS.at[k])
            pltpu.async_copy(bufs[k], gs_hbm.at[pl.ds(base, K)], semWS.at[k])
            _wait(u_hbm.at[didx.at[0]], bufd[k], semD.at[k])
            pltpu.async_copy(bufd[k], gd_hbm.at[pl.ds(base, K)], semWD.at[k])
            _wait(bufs[k], gs_hbm.at[pl.ds(base, K)], semWS.at[k])
            _wait(bufd[k], gd_hbm.at[pl.ds(base, K)], semWD.at[k])

            @pl.when(g < NGRP - 1)
            def _():
                pltpu.async_copy(u_hbm.at[sidx.at[j + NBUF]], bufs[k], semS.at[k])
                pltpu.async_copy(u_hbm.at[didx.at[j + NBUF]], bufd[k], semD.at[k])
        return carry

    lax.fori_loop(0, NGRP, group, 0)


_geom_call = pl.kernel(
    _geom_body,
    out_type=[jax.ShapeDtypeStruct((EP, 16), jnp.float32),
              jax.ShapeDtypeStruct((EP, 16), jnp.float32)],
    mesh=_MESH,
    compiler_params=_SC_PARAMS,
    scratch_types=[pltpu.VMEM((NCHUNK, K), jnp.int32),
                   pltpu.VMEM((NCHUNK, K), jnp.int32)]
                  + [pltpu.VMEM((K, 16), jnp.float32)] * NBUF
                  + [pltpu.VMEM((K, 16), jnp.float32)] * NBUF
                  + [pltpu.SemaphoreType.DMA((NBUF,))] * 4,
)


def _pq_body(a_hbm, b_hbm, src3_hbm, dst3_hbm, pq_hbm, sidx, didx,
             buf0, buf1, buf2, buf3, semA, semB, semW):
    wid = _worker_id()
    base0 = wid * PERW
    buf = [buf0, buf1, buf2, buf3]
    pltpu.sync_copy(src3_hbm.at[wid], sidx)
    pltpu.sync_copy(dst3_hbm.at[wid], didx)
    for k in range(NBUF):
        pltpu.async_copy(a_hbm.at[sidx.at[k]], buf[k], semA.at[k])
    for k in range(NBUF - 1):
        _wait(a_hbm.at[sidx.at[0]], buf[k], semA.at[k])
        pltpu.async_copy(b_hbm.at[didx.at[k]], buf[k], semB.at[k], add=True)

    def group(g, carry):
        for k in range(NBUF):
            j = g * NBUF + k
            k3 = (k + NBUF - 1) % NBUF
            base = base0 + j * K
            @pl.when(j + NBUF - 1 < NCHUNK)
            def _():
                _wait(a_hbm.at[sidx.at[0]], buf[k3], semA.at[k3])
                pltpu.async_copy(b_hbm.at[didx.at[j + NBUF - 1]], buf[k3],
                                 semB.at[k3], add=True)
            _wait(b_hbm.at[didx.at[0]], buf[k], semB.at[k])
            pltpu.async_copy(buf[k], pq_hbm.at[pl.ds(base, K)], semW.at[k])
            _wait(buf[k], pq_hbm.at[pl.ds(base, K)], semW.at[k])

            @pl.when(g < NGRP - 1)
            def _():
                pltpu.async_copy(a_hbm.at[sidx.at[j + NBUF]], buf[k], semA.at[k])
        return carry

    lax.fori_loop(0, NGRP, group, 0)


_pq_call = pl.kernel(
    _pq_body,
    out_type=jax.ShapeDtypeStruct((EP, HID), jnp.float32),
    mesh=_MESH,
    compiler_params=_SC_PARAMS,
    scratch_types=[pltpu.VMEM((NCHUNK, K), jnp.int32),
                   pltpu.VMEM((NCHUNK, K), jnp.int32)]
                  + [pltpu.VMEM((K, HID), jnp.float32)] * NBUF
                  + [pltpu.SemaphoreType.DMA((NBUF,))] * 3,
)


def _scat_body(m_hbm, dst3_hbm, zero_hbm, agg_hbm, didx,
               mbuf0, mbuf1, mbuf2, mbuf3, mbuf4, semM, semS, acc):
    cid = lax.axis_index("c")
    sid = lax.axis_index("s")
    wid = sid * NC + cid
    mbuf = [mbuf0, mbuf1, mbuf2, mbuf3, mbuf4]
    pltpu.sync_copy(zero_hbm.at[pl.ds(sid * RPS, RPS)], acc.at[pl.ds(sid * RPS, RPS)])
    plsc.subcore_barrier()
    pltpu.sync_copy(dst3_hbm.at[wid], didx)
    for k in range(2):
        base = wid * PERW + k * K
        pltpu.async_copy(m_hbm.at[pl.ds(base, K)], mbuf[k], semM.at[k])

    def group(g, carry):
        for k in range(NBUF_S):
            j = g * NBUF_S + k
            k2 = (k + 2) % NBUF_S
            _wait(m_hbm.at[pl.ds(0, K)], mbuf[k], semM.at[k])
            pltpu.async_copy(mbuf[k], acc.at[didx.at[j]], semS.at[k], add=True)
            @pl.when(j >= 3)
            def _():
                _wait(mbuf[k2], acc.at[didx.at[0]], semS.at[k2])

            @pl.when(j + 2 < NCHUNK)
            def _():
                base = wid * PERW + (j + 2) * K
                pltpu.async_copy(m_hbm.at[pl.ds(base, K)], mbuf[k2], semM.at[k2])
        return carry

    lax.fori_loop(0, NCHUNK // NBUF_S, group, 0)
    for j in range(NCHUNK - 3, NCHUNK):
        _wait(mbuf[j % NBUF_S], acc.at[didx.at[0]], semS.at[j % NBUF_S])
    plsc.subcore_barrier()
    pltpu.sync_copy(acc.at[pl.ds(sid * RPS, RPS)],
                    agg_hbm.at[cid, pl.ds(sid * RPS, RPS)])


_scat_call = pl.kernel(
    _scat_body,
    out_type=jax.ShapeDtypeStruct((NC, NP, EDGE), jnp.float32),
    mesh=_MESH,
    compiler_params=_SC_PARAMS,
    scratch_types=[pltpu.VMEM((NCHUNK, K), jnp.int32)]
                  + [pltpu.VMEM((K, EDGE), jnp.float32)] * NBUF_S
                  + [pltpu.SemaphoreType.DMA((NBUF_S,))] * 2
                  + [pltpu.VMEM_SHARED((NP, EDGE), jnp.float32)],
)


_SELQ = np.zeros((3, 16), np.float32)
_SELR = np.zeros((16, 3), np.float32)
_E3 = np.zeros((1, 16), np.float32)
for _j in range(3):
    _E3[0, _j] = 1.0
    _SELR[_j, _j] = 1.0
    for _i in range(3):
        _SELQ[_i, 3 + 3 * _i + _j] = 1.0
        _SELR[3 + 3 * _i + _j, _j] = 1.0
_ONE3 = np.ones((3, 1), np.float32)


def kernel(pos, edge_shift, lattice, params, atomic_num, edge_index, batch):
    f32 = jnp.float32
    src = edge_index[0].astype(jnp.int32)
    dst = edge_index[1].astype(jnp.int32)
    srcp = jnp.concatenate([src, jnp.full((EP - E,), N, jnp.int32)])
    dstp = jnp.concatenate([dst, jnp.full((EP - E,), N, jnp.int32)])
    src3 = srcp.reshape(NW, NCHUNK, K)
    dst3 = dstp.reshape(NW, NCHUNK, K)
    azp = jnp.pad(atomic_num.astype(jnp.int32), (0, NP - N)).reshape(NP, 1)
    batp = jnp.pad(batch.astype(jnp.int32), (0, NP - N),
                   constant_values=NG).reshape(NP, 1)
    pos16 = jnp.pad(pos.astype(f32), ((0, NP - N), (0, 13)))
    sh3 = jnp.pad(edge_shift.astype(f32), ((0, EP - E), (0, 0)))
    embp = jnp.pad(params['emb'].astype(f32), ((0, MAXZP - MAXZ), (0, 0)))
    latf = jnp.zeros((NG, 16), f32).at[:, 3:12].set(lattice.reshape(NG, 9).astype(f32))
    zeros_acc = jnp.zeros((NP, EDGE), f32)

    convs = params['convs']
    w1a = [c['We1'][:NODE] for c in convs]
    w1b = [c['We1'][NODE:2 * NODE] for c in convs]
    w1c = [c['We1'][2 * NODE:2 * NODE + 1] for c in convs]
    be1 = [c['be1'][None, :] for c in convs]
    w2 = [c['We2'] for c in convs]
    be2 = [c['be2'][None, :] for c in convs]
    wn1x = [c['Wn1'][:NODE] for c in convs]
    wn1a = [c['Wn1'][NODE:] for c in convs]
    bn1 = [c['bn1'][None, :] for c in convs]
    wn2 = [c['Wn2'] for c in convs]
    bn2 = [c['bn2'][None, :] for c in convs]
    wo1 = params['Wo1']
    bo1 = params['bo1'][None, :]
    wo2 = params['Wo2']
    bo2 = params['bo2'][None, :]

    x, a, b, u = _pre_call(azp, batp, pos16, embp, latf, w1a[0], w1b[0], be1[0])
    gs, gd = _geom_call(u, src3, dst3)
    dsq = _dsq_call(gs, gd, sh3, jnp.asarray(_SELQ), jnp.asarray(_SELR),
                    jnp.asarray(_E3), jnp.asarray(_ONE3))[0]

    for l in range(NL):
        pq = _pq_call(a, b, src3, dst3)
        m = _mid_call(pq, dsq, w1c[l], w2[l], be2[l])[0]
        agg = _scat_call(m, dst3, zeros_acc)
        if l + 1 < NL:
            wa, ba, wb = w1a[l + 1], be1[l + 1], w1b[l + 1]
        else:
            wa, ba, wb = wo1, bo1, wo1
        x, a, b = _node_call(x, agg, wn1x[l], wn1a[l], bn1[l], wn2[l], bn2[l],
                             wa, ba, wb)

    out = _read_call(a, batp, wo2, bo2)[0]
    return out


# speedup vs baseline: 1.1076x; 1.1076x over previous
import functools

import numpy as np
import jax
import jax.numpy as jnp
from jax import lax
from jax.experimental import pallas as pl
from jax.experimental.pallas import tpu as pltpu
from jax.experimental.pallas import tpu_sc as plsc

N = 10000
E = 320000
NG = 16
NODE = 128
EDGE = 64
HID = 128
NL = 3
MAXZ = 100
MAXZP = 104

NP = 10240
EP = 327680
NBLK = 1280
EBLK = 2560

NC = 2
NS = 16
NW = NC * NS
K = 128
PERW = EP // NW
NCHUNK = PERW // K
RPS = NP // NS


def _swish(v):
    return v * (1.0 / (1.0 + jnp.exp(-v)))


def _pack2(v):
    b = lax.bitcast_convert_type(v, jnp.int32)
    lo = (b[:, :64] + 0x8000) & jnp.int32(-65536)
    hi = ((b[:, 64:] + 0x8000) >> 16) & jnp.int32(0xFFFF)
    return lax.bitcast_convert_type(lo | hi, jnp.float32)


def _unpack2(p):
    b = lax.bitcast_convert_type(p, jnp.int32)
    lo = lax.bitcast_convert_type(b & jnp.int32(-65536), jnp.float32)
    hi = lax.bitcast_convert_type(b << 16, jnp.float32)
    return lo, hi


def _pre_body(az_ref, bat_ref, pos16_ref, emb_ref, latf_ref, w1a_ref, w1b_ref,
              be1_ref, x_ref, a_ref, b_ref, u_ref):
    az = az_ref[...]
    ioz = lax.broadcasted_iota(jnp.int32, (1, MAXZP), 1)
    oh = (az == ioz).astype(jnp.float32)
    x = jnp.dot(oh, emb_ref[...], preferred_element_type=jnp.float32)
    x_ref[...] = x
    a_ref[...] = _pack2(jnp.dot(x, w1a_ref[...], preferred_element_type=jnp.float32)
                        + be1_ref[...])
    b_ref[...] = _pack2(jnp.dot(x, w1b_ref[...], preferred_element_type=jnp.float32))
    bat = bat_ref[...]
    iog = lax.broadcasted_iota(jnp.int32, (1, NG), 1)
    ohg = (bat == iog).astype(jnp.float32)
    u_ref[...] = pos16_ref[...] + jnp.dot(ohg, latf_ref[...],
                                          preferred_element_type=jnp.float32)


def _dsq_body(gs_ref, gd_ref, sh_ref, selq_ref, selr_ref, e3_ref, one3_ref,
              dsq_ref):
    gs = gs_ref[...]
    gd = gd_ref[...]
    shb = jnp.dot(sh_ref[...], selq_ref[...],
                  preferred_element_type=jnp.float32)
    full = (gd - gs) * e3_ref[...] + shb * gs
    ev = jnp.dot(full, selr_ref[...], preferred_element_type=jnp.float32)
    dsq_ref[...] = jnp.dot(ev * ev, one3_ref[...], preferred_element_type=jnp.float32)


def _mid_body(pa_ref, pb_ref, dsq_ref, w1c_ref, w2_ref, be2_ref, m_ref):
    a_lo, a_hi = _unpack2(pa_ref[...])
    b_lo, b_hi = _unpack2(pb_ref[...])
    dsq = dsq_ref[...]
    z_lo = _swish(a_lo + b_lo + dsq * w1c_ref[:, :64])
    z_hi = _swish(a_hi + b_hi + dsq * w1c_ref[:, 64:])
    m = (jnp.dot(z_lo, w2_ref[:64, :], preferred_element_type=jnp.float32)
         + jnp.dot(z_hi, w2_ref[64:, :], preferred_element_type=jnp.float32)
         + be2_ref[...])
    m_ref[...] = _swish(m)


def _node_body(x_ref, agg_ref, wn1x_ref, wn1a_ref, bn1_ref, wn2_ref, bn2_ref,
               wa_ref, ba_ref, wb_ref, xn_ref, a_ref, b_ref):
    agg = agg_ref[0] + agg_ref[1]
    h = _swish(jnp.dot(x_ref[...], wn1x_ref[...], preferred_element_type=jnp.float32)
               + jnp.dot(agg, wn1a_ref[...], preferred_element_type=jnp.float32)
               + bn1_ref[...])
    xn = x_ref[...] + jnp.dot(h, wn2_ref[...], preferred_element_type=jnp.float32) + bn2_ref[...]
    xn_ref[...] = xn
    a_ref[...] = _pack2(jnp.dot(xn, wa_ref[...], preferred_element_type=jnp.float32)
                        + ba_ref[...])
    b_ref[...] = _pack2(jnp.dot(xn, wb_ref[...], preferred_element_type=jnp.float32))


def _read_body(a2_ref, bat_ref, wo2_ref, bo2_ref, out_ref):
    a_lo, a_hi = _unpack2(a2_ref[...])
    o = (jnp.dot(_swish(a_lo), wo2_ref[:64, :], preferred_element_type=jnp.float32)
         + jnp.dot(_swish(a_hi), wo2_ref[64:, :], preferred_element_type=jnp.float32)
         + bo2_ref[...])
    iog = lax.broadcasted_iota(jnp.int32, (1, NG), 1)
    msk = (bat_ref[...] == iog).astype(jnp.float32)
    contrib = lax.dot_general(msk, o, (((0,), (0,)), ((), ())),
                              preferred_element_type=jnp.float32)
    @pl.when(pl.program_id(0) == 0)
    def _():
        out_ref[...] = jnp.zeros_like(out_ref)
    out_ref[...] += contrib


def _full(shape):
    return pl.BlockSpec(shape, lambda i: tuple(0 for _ in shape))


def _rows(shape):
    nd = len(shape)
    if nd == 2:
        return pl.BlockSpec(shape, lambda i: (i, 0))
    return pl.BlockSpec(shape, lambda i: (0, i, 0))


_pre_call = pl.pallas_call(
    _pre_body,
    grid=(NP // NBLK,),
    in_specs=[_rows((NBLK, 1)), _rows((NBLK, 1)), _rows((NBLK, 16)),
              _full((MAXZP, NODE)), _full((NG, 16)), _full((NODE, HID)),
              _full((NODE, HID)), _full((1, HID))],
    out_specs=[_rows((NBLK, NODE)), _rows((NBLK, 64)), _rows((NBLK, 64)),
               _rows((NBLK, 16))],
    out_shape=[jax.ShapeDtypeStruct((NP, NODE), jnp.float32),
               jax.ShapeDtypeStruct((NP, 64), jnp.float32),
               jax.ShapeDtypeStruct((NP, 64), jnp.float32),
               jax.ShapeDtypeStruct((NP, 16), jnp.float32)],
)

_dsq_call = pl.pallas_call(
    _dsq_body,
    grid=(EP // EBLK,),
    in_specs=[_rows((EBLK, 16)), _rows((EBLK, 16)), _rows((EBLK, 3)),
              _full((3, 16)), _full((16, 3)), _full((1, 16)), _full((3, 1))],
    out_specs=[_rows((EBLK, 1))],
    out_shape=[jax.ShapeDtypeStruct((EP, 1), jnp.float32)],
)

_mid_call = pl.pallas_call(
    _mid_body,
    grid=(EP // EBLK,),
    in_specs=[_rows((EBLK, 64)), _rows((EBLK, 64)), _rows((EBLK, 1)),
              _full((1, HID)), _full((HID, EDGE)), _full((1, EDGE))],
    out_specs=[_rows((EBLK, EDGE))],
    out_shape=[jax.ShapeDtypeStruct((EP, EDGE), jnp.float32)],
)

_node_call = pl.pallas_call(
    _node_body,
    grid=(NP // NBLK,),
    in_specs=[_rows((NBLK, NODE)), _rows((NC, NBLK, EDGE)),
              _full((NODE, HID)), _full((EDGE, HID)), _full((1, HID)),
              _full((HID, NODE)), _full((1, NODE)),
              _full((NODE, HID)), _full((1, HID)), _full((NODE, HID))],
    out_specs=[_rows((NBLK, NODE)), _rows((NBLK, 64)), _rows((NBLK, 64))],
    out_shape=[jax.ShapeDtypeStruct((NP, NODE), jnp.float32),
               jax.ShapeDtypeStruct((NP, 64), jnp.float32),
               jax.ShapeDtypeStruct((NP, 64), jnp.float32)],
)

_read_call = pl.pallas_call(
    _read_body,
    grid=(NP // NBLK,),
    in_specs=[_rows((NBLK, 64)), _rows((NBLK, 1)), _full((HID, 1)),
              _full((1, 1))],
    out_specs=[_full((NG, 1))],
    out_shape=[jax.ShapeDtypeStruct((NG, 1), jnp.float32)],
)


_MESH = plsc.VectorSubcoreMesh(core_axis_name="c", subcore_axis_name="s")
_SC_PARAMS = pltpu.CompilerParams(use_tc_tiling_on_sc=False)


def _worker_id():
    return lax.axis_index("s") * NC + lax.axis_index("c")


NBUF = 4
NBUF_S = 5
NGRP = NCHUNK // NBUF


def _wait(src, dst, sem):
    pltpu.make_async_copy(src, dst, sem).wait()


def _make_gather2(d):
    def body(t1_hbm, t2_hbm, src3_hbm, dst3_hbm, o1_hbm, o2_hbm, sidx, didx,
             b10, b11, b12, b13, b20, b21, b22, b23, semS, semD, semWS, semWD):
        wid = _worker_id()
        base0 = wid * PERW
        bufs = [b10, b11, b12, b13]
        bufd = [b20, b21, b22, b23]
        pltpu.sync_copy(src3_hbm.at[wid], sidx)
        pltpu.sync_copy(dst3_hbm.at[wid], didx)
        for k in range(NBUF):
            pltpu.async_copy(t1_hbm.at[sidx.at[k]], bufs[k], semS.at[k])
            pltpu.async_copy(t2_hbm.at[didx.at[k]], bufd[k], semD.at[k])

        def group(g, carry):
            for k in range(NBUF):
                j = g * NBUF + k
                base = base0 + j * K
                _wait(t1_hbm.at[sidx.at[0]], bufs[k], semS.at[k])
                pltpu.async_copy(bufs[k], o1_hbm.at[pl.ds(base, K)], semWS.at[k])
                _wait(t2_hbm.at[didx.at[0]], bufd[k], semD.at[k])
                pltpu.async_copy(bufd[k], o2_hbm.at[pl.ds(base, K)], semWD.at[k])
                _wait(bufs[k], o1_hbm.at[pl.ds(base, K)], semWS.at[k])
                _wait(bufd[k], o2_hbm.at[pl.ds(base, K)], semWD.at[k])

                @pl.when(g < NGRP - 1)
                def _():
                    pltpu.async_copy(t1_hbm.at[sidx.at[j + NBUF]], bufs[k], semS.at[k])
                    pltpu.async_copy(t2_hbm.at[didx.at[j + NBUF]], bufd[k], semD.at[k])
            return carry

        lax.fori_loop(0, NGRP, group, 0)

    return pl.kernel(
        body,
        out_type=[jax.ShapeDtypeStruct((EP, d), jnp.float32),
                  jax.ShapeDtypeStruct((EP, d), jnp.float32)],
        mesh=_MESH,
        compiler_params=_SC_PARAMS,
        scratch_types=[pltpu.VMEM((NCHUNK, K), jnp.int32),
                       pltpu.VMEM((NCHUNK, K), jnp.int32)]
                      + [pltpu.VMEM((K, d), jnp.float32)] * (2 * NBUF)
                      + [pltpu.SemaphoreType.DMA((NBUF,))] * 4,
    )


_geom_call = _make_gather2(16)
_ab_call = _make_gather2(64)


def _scat_body(m_hbm, dst3_hbm, zero_hbm, agg_hbm, didx,
               mbuf0, mbuf1, mbuf2, mbuf3, mbuf4, semM, semS, acc):
    cid = lax.axis_index("c")
    sid = lax.axis_index("s")
    wid = sid * NC + cid
    mbuf = [mbuf0, mbuf1, mbuf2, mbuf3, mbuf4]
    pltpu.sync_copy(zero_hbm.at[pl.ds(sid * RPS, RPS)], acc.at[pl.ds(sid * RPS, RPS)])
    plsc.subcore_barrier()
    pltpu.sync_copy(dst3_hbm.at[wid], didx)
    for k in range(2):
        base = wid * PERW + k * K
        pltpu.async_copy(m_hbm.at[pl.ds(base, K)], mbuf[k], semM.at[k])

    def group(g, carry):
        for k in range(NBUF_S):
            j = g * NBUF_S + k
            k2 = (k + 2) % NBUF_S
            _wait(m_hbm.at[pl.ds(0, K)], mbuf[k], semM.at[k])
            pltpu.async_copy(mbuf[k], acc.at[didx.at[j]], semS.at[k], add=True)
            @pl.when(j >= 3)
            def _():
                _wait(mbuf[k2], acc.at[didx.at[0]], semS.at[k2])

            @pl.when(j + 2 < NCHUNK)
            def _():
                base = wid * PERW + (j + 2) * K
                pltpu.async_copy(m_hbm.at[pl.ds(base, K)], mbuf[k2], semM.at[k2])
        return carry

    lax.fori_loop(0, NCHUNK // NBUF_S, group, 0)
    for j in range(NCHUNK - 3, NCHUNK):
        _wait(mbuf[j % NBUF_S], acc.at[didx.at[0]], semS.at[j % NBUF_S])
    plsc.subcore_barrier()
    pltpu.sync_copy(acc.at[pl.ds(sid * RPS, RPS)],
                    agg_hbm.at[cid, pl.ds(sid * RPS, RPS)])


_scat_call = pl.kernel(
    _scat_body,
    out_type=jax.ShapeDtypeStruct((NC, NP, EDGE), jnp.float32),
    mesh=_MESH,
    compiler_params=_SC_PARAMS,
    scratch_types=[pltpu.VMEM((NCHUNK, K), jnp.int32)]
                  + [pltpu.VMEM((K, EDGE), jnp.float32)] * NBUF_S
                  + [pltpu.SemaphoreType.DMA((NBUF_S,))] * 2
                  + [pltpu.VMEM_SHARED((NP, EDGE), jnp.float32)],
)


_SELQ = np.zeros((3, 16), np.float32)
_SELR = np.zeros((16, 3), np.float32)
_E3 = np.zeros((1, 16), np.float32)
for _j in range(3):
    _E3[0, _j] = 1.0
    _SELR[_j, _j] = 1.0
    for _i in range(3):
        _SELQ[_i, 3 + 3 * _i + _j] = 1.0
        _SELR[3 + 3 * _i + _j, _j] = 1.0
_ONE3 = np.ones((3, 1), np.float32)


def kernel(pos, edge_shift, lattice, params, atomic_num, edge_index, batch):
    f32 = jnp.float32
    src = edge_index[0].astype(jnp.int32)
    dst = edge_index[1].astype(jnp.int32)
    srcp = jnp.concatenate([src, jnp.full((EP - E,), N, jnp.int32)])
    dstp = jnp.concatenate([dst, jnp.full((EP - E,), N, jnp.int32)])
    src3 = srcp.reshape(NW, NCHUNK, K)
    dst3 = dstp.reshape(NW, NCHUNK, K)
    azp = jnp.pad(atomic_num.astype(jnp.int32), (0, NP - N)).reshape(NP, 1)
    batp = jnp.pad(batch.astype(jnp.int32), (0, NP - N),
                   constant_values=NG).reshape(NP, 1)
    pos16 = jnp.pad(pos.astype(f32), ((0, NP - N), (0, 13)))
    sh3 = jnp.pad(edge_shift.astype(f32), ((0, EP - E), (0, 0)))
    embp = jnp.pad(params['emb'].astype(f32), ((0, MAXZP - MAXZ), (0, 0)))
    latf = jnp.zeros((NG, 16), f32).at[:, 3:12].set(lattice.reshape(NG, 9).astype(f32))
    zeros_acc = jnp.zeros((NP, EDGE), f32)

    convs = params['convs']
    w1a = [c['We1'][:NODE] for c in convs]
    w1b = [c['We1'][NODE:2 * NODE] for c in convs]
    w1c = [c['We1'][2 * NODE:2 * NODE + 1] for c in convs]
    be1 = [c['be1'][None, :] for c in convs]
    w2 = [c['We2'] for c in convs]
    be2 = [c['be2'][None, :] for c in convs]
    wn1x = [c['Wn1'][:NODE] for c in convs]
    wn1a = [c['Wn1'][NODE:] for c in convs]
    bn1 = [c['bn1'][None, :] for c in convs]
    wn2 = [c['Wn2'] for c in convs]
    bn2 = [c['bn2'][None, :] for c in convs]
    wo1 = params['Wo1']
    bo1 = params['bo1'][None, :]
    wo2 = params['Wo2']
    bo2 = params['bo2'][None, :]

    x, a, b, u = _pre_call(azp, batp, pos16, embp, latf, w1a[0], w1b[0], be1[0])
    gs, gd = _geom_call(u, u, src3, dst3)
    dsq = _dsq_call(gs, gd, sh3, jnp.asarray(_SELQ), jnp.asarray(_SELR),
                    jnp.asarray(_E3), jnp.asarray(_ONE3))[0]

    for l in range(NL):
        pa, pb = _ab_call(a, b, src3, dst3)
        m = _mid_call(pa, pb, dsq, w1c[l], w2[l], be2[l])[0]
        agg = _scat_call(m, dst3, zeros_acc)
        if l + 1 < NL:
            wa, ba, wb = w1a[l + 1], be1[l + 1], w1b[l + 1]
        else:
            wa, ba, wb = wo1, bo1, wo1
        x, a, b = _node_call(x, agg, wn1x[l], wn1a[l], bn1[l], wn2[l], bn2[l],
                             wa, ba, wb)

    out = _read_call(a, batp, wo2, bo2)[0]
    return out


# speedup vs baseline: 1.1451x; 1.0339x over previous
import numpy as np
import jax
import jax.numpy as jnp
from jax import lax
from jax.experimental import pallas as pl
from jax.experimental.pallas import tpu as pltpu
from jax.experimental.pallas import tpu_sc as plsc

N = 10000
E = 320000
NG = 16
NODE = 128
EDGE = 64
HID = 128
NL = 3
MAXZ = 100
MAXZP = 104

NP = 10240
EP = 327680
NBLK = 1280
EBLK = 2560

NC = 2
NS = 16
NW = NC * NS
K = 128
PERW = EP // NW
NCHUNK = PERW // K
RPS = NP // NS
NBUF = 4
NGRP = NCHUNK // NBUF
NBUF_P = 5
NBUF_S = 5


def _swish(v):
    return v * (1.0 / (1.0 + jnp.exp(-v)))


def _pre_body(az_ref, bat_ref, pos16_ref, emb_ref, latf_ref, w1a_ref, w1b_ref,
              be1_ref, x_ref, a_ref, b_ref, u_ref):
    az = az_ref[...]
    ioz = lax.broadcasted_iota(jnp.int32, (1, MAXZP), 1)
    oh = (az == ioz).astype(jnp.float32)
    x = jnp.dot(oh, emb_ref[...], preferred_element_type=jnp.float32)
    x_ref[...] = x
    a_ref[...] = jnp.dot(x, w1a_ref[...], preferred_element_type=jnp.float32) + be1_ref[...]
    b_ref[...] = jnp.dot(x, w1b_ref[...], preferred_element_type=jnp.float32)
    bat = bat_ref[...]
    iog = lax.broadcasted_iota(jnp.int32, (1, NG), 1)
    ohg = (bat == iog).astype(jnp.float32)
    u_ref[...] = pos16_ref[...] + jnp.dot(ohg, latf_ref[...],
                                          preferred_element_type=jnp.float32)


def _dsq_body(gs_ref, gd_ref, sh_ref, selq_ref, selr_ref, e3_ref, one3_ref,
              dsq_ref):
    gs = gs_ref[...]
    gd = gd_ref[...]
    shb = jnp.dot(sh_ref[...], selq_ref[...],
                  preferred_element_type=jnp.float32)
    full = (gd - gs) * e3_ref[...] + shb * gs
    ev = jnp.dot(full, selr_ref[...], preferred_element_type=jnp.float32)
    dsq_ref[...] = jnp.dot(ev * ev, one3_ref[...], preferred_element_type=jnp.float32)


def _mid_body(pq_ref, dsq_ref, w1c_ref, w2_ref, be2_ref, m_ref):
    z = _swish(pq_ref[...] + dsq_ref[...] * w1c_ref[...])
    m = jnp.dot(z, w2_ref[...], preferred_element_type=jnp.float32) + be2_ref[...]
    m_ref[...] = _swish(m)


def _node_body(x_ref, agg_ref, wn1x_ref, wn1a_ref, bn1_ref, wn2_ref, bn2_ref,
               wa_ref, ba_ref, wb_ref, xn_ref, a_ref, b_ref):
    agg = agg_ref[0] + agg_ref[1]
    h = _swish(jnp.dot(x_ref[...], wn1x_ref[...], preferred_element_type=jnp.float32)
               + jnp.dot(agg, wn1a_ref[...], preferred_element_type=jnp.float32)
               + bn1_ref[...])
    xn = x_ref[...] + jnp.dot(h, wn2_ref[...], preferred_element_type=jnp.float32) + bn2_ref[...]
    xn_ref[...] = xn
    a_ref[...] = jnp.dot(xn, wa_ref[...], preferred_element_type=jnp.float32) + ba_ref[...]
    b_ref[...] = jnp.dot(xn, wb_ref[...], preferred_element_type=jnp.float32)


def _read_body(a2_ref, bat_ref, wo2_ref, bo2_ref, out_ref):
    h = _swish(a2_ref[...])
    o = jnp.dot(h, wo2_ref[...], preferred_element_type=jnp.float32) + bo2_ref[...]
    iog = lax.broadcasted_iota(jnp.int32, (1, NG), 1)
    msk = (bat_ref[...] == iog).astype(jnp.float32)
    contrib = lax.dot_general(msk, o, (((0,), (0,)), ((), ())),
                              preferred_element_type=jnp.float32)
    @pl.when(pl.program_id(0) == 0)
    def _():
        out_ref[...] = jnp.zeros_like(out_ref)
    out_ref[...] += contrib


def _full(shape):
    return pl.BlockSpec(shape, lambda i: tuple(0 for _ in shape))


def _rows(shape):
    if len(shape) == 2:
        return pl.BlockSpec(shape, lambda i: (i, 0))
    return pl.BlockSpec(shape, lambda i: (0, i, 0))


_pre_call = pl.pallas_call(
    _pre_body,
    grid=(NP // NBLK,),
    in_specs=[_rows((NBLK, 1)), _rows((NBLK, 1)), _rows((NBLK, 16)),
              _full((MAXZP, NODE)), _full((NG, 16)), _full((NODE, HID)),
              _full((NODE, HID)), _full((1, HID))],
    out_specs=[_rows((NBLK, NODE)), _rows((NBLK, HID)), _rows((NBLK, HID)),
               _rows((NBLK, 16))],
    out_shape=[jax.ShapeDtypeStruct((NP, NODE), jnp.float32),
               jax.ShapeDtypeStruct((NP, HID), jnp.float32),
               jax.ShapeDtypeStruct((NP, HID), jnp.float32),
               jax.ShapeDtypeStruct((NP, 16), jnp.float32)],
)

_dsq_call = pl.pallas_call(
    _dsq_body,
    grid=(EP // EBLK,),
    in_specs=[_rows((EBLK, 16)), _rows((EBLK, 16)), _rows((EBLK, 3)),
              _full((3, 16)), _full((16, 3)), _full((1, 16)), _full((3, 1))],
    out_specs=[_rows((EBLK, 1))],
    out_shape=[jax.ShapeDtypeStruct((EP, 1), jnp.float32)],
)

_mid_call = pl.pallas_call(
    _mid_body,
    grid=(EP // EBLK,),
    in_specs=[_rows((EBLK, HID)), _rows((EBLK, 1)), _full((1, HID)),
              _full((HID, EDGE)), _full((1, EDGE))],
    out_specs=[_rows((EBLK, EDGE))],
    out_shape=[jax.ShapeDtypeStruct((EP, EDGE), jnp.float32)],
)

_node_call = pl.pallas_call(
    _node_body,
    grid=(NP // NBLK,),
    in_specs=[_rows((NBLK, NODE)), _rows((NC, NBLK, EDGE)),
              _full((NODE, HID)), _full((EDGE, HID)), _full((1, HID)),
              _full((HID, NODE)), _full((1, NODE)),
              _full((NODE, HID)), _full((1, HID)), _full((NODE, HID))],
    out_specs=[_rows((NBLK, NODE)), _rows((NBLK, HID)), _rows((NBLK, HID))],
    out_shape=[jax.ShapeDtypeStruct((NP, NODE), jnp.float32),
               jax.ShapeDtypeStruct((NP, HID), jnp.float32),
               jax.ShapeDtypeStruct((NP, HID), jnp.float32)],
)

_read_call = pl.pallas_call(
    _read_body,
    grid=(NP // NBLK,),
    in_specs=[_rows((NBLK, HID)), _rows((NBLK, 1)), _full((HID, 1)),
              _full((1, 1))],
    out_specs=[_full((NG, 1))],
    out_shape=[jax.ShapeDtypeStruct((NG, 1), jnp.float32)],
)


_MESH = plsc.VectorSubcoreMesh(core_axis_name="c", subcore_axis_name="s")
_SC_PARAMS = pltpu.CompilerParams(use_tc_tiling_on_sc=False)


def _worker_id():
    return lax.axis_index("s") * NC + lax.axis_index("c")


def _wait(src, dst, sem):
    pltpu.make_async_copy(src, dst, sem).wait()


def _geom_body(u1_hbm, u2_hbm, src3_hbm, dst3_hbm, gs_hbm, gd_hbm, sidx, didx,
               bs0, bs1, bs2, bs3, bd0, bd1, bd2, bd3,
               semS, semD, semWS, semWD):
    wid = _worker_id()
    base0 = wid * PERW
    bufs = [bs0, bs1, bs2, bs3]
    bufd = [bd0, bd1, bd2, bd3]
    pltpu.sync_copy(src3_hbm.at[wid], sidx)
    pltpu.sync_copy(dst3_hbm.at[wid], didx)
    for k in range(NBUF):
        pltpu.async_copy(u1_hbm.at[sidx.at[k]], bufs[k], semS.at[k])
        pltpu.async_copy(u2_hbm.at[didx.at[k]], bufd[k], semD.at[k])

    def group(g, carry):
        for k in range(NBUF):
            j = g * NBUF + k
            base = base0 + j * K
            _wait(u1_hbm.at[sidx.at[0]], bufs[k], semS.at[k])
            pltpu.async_copy(bufs[k], gs_hbm.at[pl.ds(base, K)], semWS.at[k])
            _wait(u2_hbm.at[didx.at[0]], bufd[k], semD.at[k])
            pltpu.async_copy(bufd[k], gd_hbm.at[pl.ds(base, K)], semWD.at[k])
            _wait(bufs[k], gs_hbm.at[pl.ds(base, K)], semWS.at[k])
            _wait(bufd[k], gd_hbm.at[pl.ds(base, K)], semWD.at[k])

            @pl.when(g < NGRP - 1)
            def _():
                pltpu.async_copy(u1_hbm.at[sidx.at[j + NBUF]], bufs[k], semS.at[k])
                pltpu.async_copy(u2_hbm.at[didx.at[j + NBUF]], bufd[k], semD.at[k])
        return carry

    lax.fori_loop(0, NGRP, group, 0)


_geom_call = pl.kernel(
    _geom_body,
    out_type=[jax.ShapeDtypeStruct((EP, 16), jnp.float32),
              jax.ShapeDtypeStruct((EP, 16), jnp.float32)],
    mesh=_MESH,
    compiler_params=_SC_PARAMS,
    scratch_types=[pltpu.VMEM((NCHUNK, K), jnp.int32),
                   pltpu.VMEM((NCHUNK, K), jnp.int32)]
                  + [pltpu.VMEM((K, 16), jnp.float32)] * (2 * NBUF)
                  + [pltpu.SemaphoreType.DMA((NBUF,))] * 4,
)


def _pq_body(a_hbm, b_hbm, src3_hbm, dst3_hbm, pq_hbm, sidx, didx,
             buf0, buf1, buf2, buf3, buf4, semA, semB, semW):
    wid = _worker_id()
    base0 = wid * PERW
    buf = [buf0, buf1, buf2, buf3, buf4]
    pltpu.sync_copy(src3_hbm.at[wid], sidx)
    pltpu.sync_copy(dst3_hbm.at[wid], didx)
    for k in range(NBUF_P):
        pltpu.async_copy(a_hbm.at[sidx.at[k]], buf[k], semA.at[k])
    for k in range(2):
        _wait(a_hbm.at[sidx.at[0]], buf[k], semA.at[k])
        pltpu.async_copy(b_hbm.at[didx.at[k]], buf[k], semB.at[k], add=True)

    def group(g, carry):
        for k in range(NBUF_P):
            j = g * NBUF_P + k
            k2 = (k + 2) % NBUF_P
            base = base0 + j * K
            @pl.when(j + 2 < NCHUNK)
            def _():
                _wait(a_hbm.at[sidx.at[0]], buf[k2], semA.at[k2])
                pltpu.async_copy(b_hbm.at[didx.at[j + 2]], buf[k2],
                                 semB.at[k2], add=True)
            _wait(b_hbm.at[didx.at[0]], buf[k], semB.at[k])
            pltpu.async_copy(buf[k], pq_hbm.at[pl.ds(base, K)], semW.at[k])
            _wait(buf[k], pq_hbm.at[pl.ds(base, K)], semW.at[k])

            @pl.when(j + NBUF_P < NCHUNK)
            def _():
                pltpu.async_copy(a_hbm.at[sidx.at[j + NBUF_P]], buf[k], semA.at[k])
        return carry

    lax.fori_loop(0, NCHUNK // NBUF_P, group, 0)


_pq_call = pl.kernel(
    _pq_body,
    out_type=jax.ShapeDtypeStruct((EP, HID), jnp.float32),
    mesh=_MESH,
    compiler_params=_SC_PARAMS,
    scratch_types=[pltpu.VMEM((NCHUNK, K), jnp.int32),
                   pltpu.VMEM((NCHUNK, K), jnp.int32)]
                  + [pltpu.VMEM((K, HID), jnp.float32)] * NBUF_P
                  + [pltpu.SemaphoreType.DMA((NBUF_P,))] * 3,
)


def _scat_body(m_hbm, dst3_hbm, zero_hbm, agg_hbm, didx,
               mbuf0, mbuf1, mbuf2, mbuf3, mbuf4, semM, semS, acc):
    cid = lax.axis_index("c")
    sid = lax.axis_index("s")
    wid = sid * NC + cid
    mbuf = [mbuf0, mbuf1, mbuf2, mbuf3, mbuf4]
    pltpu.sync_copy(zero_hbm.at[pl.ds(sid * RPS, RPS)], acc.at[pl.ds(sid * RPS, RPS)])
    plsc.subcore_barrier()
    pltpu.sync_copy(dst3_hbm.at[wid], didx)
    for k in range(2):
        base = wid * PERW + k * K
        pltpu.async_copy(m_hbm.at[pl.ds(base, K)], mbuf[k], semM.at[k])

    def group(g, carry):
        for k in range(NBUF_S):
            j = g * NBUF_S + k
            k2 = (k + 2) % NBUF_S
            _wait(m_hbm.at[pl.ds(0, K)], mbuf[k], semM.at[k])
            pltpu.async_copy(mbuf[k], acc.at[didx.at[j]], semS.at[k], add=True)
            @pl.when(j >= 3)
            def _():
                _wait(mbuf[k2], acc.at[didx.at[0]], semS.at[k2])

            @pl.when(j + 2 < NCHUNK)
            def _():
                base = wid * PERW + (j + 2) * K
                pltpu.async_copy(m_hbm.at[pl.ds(base, K)], mbuf[k2], semM.at[k2])
        return carry

    lax.fori_loop(0, NCHUNK // NBUF_S, group, 0)
    for j in range(NCHUNK - 3, NCHUNK):
        _wait(mbuf[j % NBUF_S], acc.at[didx.at[0]], semS.at[j % NBUF_S])
    plsc.subcore_barrier()
    pltpu.sync_copy(acc.at[pl.ds(sid * RPS, RPS)],
                    agg_hbm.at[cid, pl.ds(sid * RPS, RPS)])


_scat_call = pl.kernel(
    _scat_body,
    out_type=jax.ShapeDtypeStruct((NC, NP, EDGE), jnp.float32),
    mesh=_MESH,
    compiler_params=_SC_PARAMS,
    scratch_types=[pltpu.VMEM((NCHUNK, K), jnp.int32)]
                  + [pltpu.VMEM((K, EDGE), jnp.float32)] * NBUF_S
                  + [pltpu.SemaphoreType.DMA((NBUF_S,))] * 2
                  + [pltpu.VMEM_SHARED((NP, EDGE), jnp.float32)],
)


_SELQ = np.zeros((3, 16), np.float32)
_SELR = np.zeros((16, 3), np.float32)
_E3 = np.zeros((1, 16), np.float32)
for _j in range(3):
    _E3[0, _j] = 1.0
    _SELR[_j, _j] = 1.0
    for _i in range(3):
        _SELQ[_i, 3 + 3 * _i + _j] = 1.0
        _SELR[3 + 3 * _i + _j, _j] = 1.0
_ONE3 = np.ones((3, 1), np.float32)


def kernel(pos, edge_shift, lattice, params, atomic_num, edge_index, batch):
    f32 = jnp.float32
    src = edge_index[0].astype(jnp.int32)
    dst = edge_index[1].astype(jnp.int32)
    srcp = jnp.concatenate([src, jnp.full((EP - E,), N, jnp.int32)])
    dstp = jnp.concatenate([dst, jnp.full((EP - E,), N, jnp.int32)])
    src3 = srcp.reshape(NW, NCHUNK, K)
    dst3 = dstp.reshape(NW, NCHUNK, K)
    azp = jnp.pad(atomic_num.astype(jnp.int32), (0, NP - N)).reshape(NP, 1)
    batp = jnp.pad(batch.astype(jnp.int32), (0, NP - N),
                   constant_values=NG).reshape(NP, 1)
    pos16 = jnp.pad(pos.astype(f32), ((0, NP - N), (0, 13)))
    sh3 = jnp.pad(edge_shift.astype(f32), ((0, EP - E), (0, 0)))
    embp = jnp.pad(params['emb'].astype(f32), ((0, MAXZP - MAXZ), (0, 0)))
    latf = jnp.zeros((NG, 16), f32).at[:, 3:12].set(lattice.reshape(NG, 9).astype(f32))
    zeros_acc = jnp.zeros((NP, EDGE), f32)

    convs = params['convs']
    w1a = [c['We1'][:NODE] for c in convs]
    w1b = [c['We1'][NODE:2 * NODE] for c in convs]
    w1c = [c['We1'][2 * NODE:2 * NODE + 1] for c in convs]
    be1 = [c['be1'][None, :] for c in convs]
    w2 = [c['We2'] for c in convs]
    be2 = [c['be2'][None, :] for c in convs]
    wn1x = [c['Wn1'][:NODE] for c in convs]
    wn1a = [c['Wn1'][NODE:] for c in convs]
    bn1 = [c['bn1'][None, :] for c in convs]
    wn2 = [c['Wn2'] for c in convs]
    bn2 = [c['bn2'][None, :] for c in convs]
    wo1 = params['Wo1']
    bo1 = params['bo1'][None, :]
    wo2 = params['Wo2']
    bo2 = params['bo2'][None, :]

    x, a, b, u = _pre_call(azp, batp, pos16, embp, latf, w1a[0], w1b[0], be1[0])
    gs, gd = _geom_call(u, u, src3, dst3)
    dsq = _dsq_call(gs, gd, sh3, jnp.asarray(_SELQ), jnp.asarray(_SELR),
                    jnp.asarray(_E3), jnp.asarray(_ONE3))[0]

    for l in range(NL):
        pq = _pq_call(a, b, src3, dst3)
        m = _mid_call(pq, dsq, w1c[l], w2[l], be2[l])[0]
        agg = _scat_call(m, dst3, zeros_acc)
        if l + 1 < NL:
            wa, ba, wb = w1a[l + 1], be1[l + 1], w1b[l + 1]
        else:
            wa, ba, wb = wo1, bo1, wo1
        x, a, b = _node_call(x, agg, wn1x[l], wn1a[l], bn1[l], wn2[l], bn2[l],
                             wa, ba, wb)

    out = _read_call(a, batp, wo2, bo2)[0]
    return out
